# Initial kernel scaffold; baseline (speedup 1.0000x reference)
import jax
import jax.numpy as jnp
from jax.experimental import pallas as pl


def kernel(x, edge_index, W1, b1, Wmu, bmu, Wls, bls):
    raise NotImplementedError("write your pallas kernel here")


# speedup vs baseline: 17.3060x; 17.3060x over previous
import functools

import jax
import jax.numpy as jnp
from jax import lax
from jax.experimental import pallas as pl
from jax.experimental.pallas import tpu as pltpu
from jax.experimental.pallas import tpu_sc as plsc

_N = 10000
_E = 320000
_D = 128
_NC = 2
_NS = 16
_NW = _NC * _NS
_EPW = _E // _NW
_K = 80
_NCH = _EPW // _K
_NP = 10240
_RPT = _NP // _NS
_RCH = 32

_mesh = plsc.VectorSubcoreMesh(core_axis_name="c", subcore_axis_name="s")


@functools.partial(
    pl.kernel,
    out_type=jax.ShapeDtypeStruct((_NW, _N), jnp.float32),
    mesh=_mesh,
    scratch_types=[
        pltpu.VMEM((_EPW,), jnp.int32),
        pltpu.VMEM((_N,), jnp.float32),
    ],
    compiler_params=pltpu.CompilerParams(needs_layout_passes=False),
)
def _sc_deg(dst_hbm, out_hbm, dst_v, hist_v):
    cid = lax.axis_index("c")
    sid = lax.axis_index("s")
    wid = sid * _NC + cid
    pltpu.sync_copy(dst_hbm.at[pl.ds(wid * _EPW, _EPW)], dst_v)
    zeros = jnp.zeros((16,), jnp.float32)
    ones = jnp.ones((16,), jnp.float32)

    def zbody(i, c):
        hist_v[pl.ds(i * 16, 16)] = zeros
        return c

    lax.fori_loop(0, _N // 16, zbody, 0)

    def hbody(i, c):
        idx = dst_v[pl.ds(i * 16, 16)]
        plsc.addupdate_scatter(hist_v, [idx], ones)
        return c

    lax.fori_loop(0, _EPW // 16, hbody, 0)
    pltpu.sync_copy(hist_v, out_hbm.at[wid])


@functools.partial(
    pl.kernel,
    out_type=jax.ShapeDtypeStruct((_NC, _NP, _D), jnp.float32),
    mesh=_mesh,
    scratch_types=[
        pltpu.VMEM((_K,), jnp.int32),
        pltpu.VMEM((_K,), jnp.int32),
        pltpu.VMEM((_K, _D), jnp.float32),
        pltpu.VMEM((_RCH, _D), jnp.float32),
        pltpu.VMEM_SHARED((_NP, _D), jnp.float32),
        pltpu.SemaphoreType.DMA,
    ],
)
def _sc_spmm(xs_hbm, src_hbm, dst_hbm, out_hbm, si_v, di_v, rows_v, buf_v,
             acc_sh, sem):
    cid = lax.axis_index("c")
    sid = lax.axis_index("s")
    wid = sid * _NC + cid
    zeros = jnp.zeros((16,), jnp.float32)
    for r in range(_RCH):
        for j in range(_D // 16):
            buf_v[r, pl.ds(j * 16, 16)] = zeros

    def zb(i, c):
        pltpu.sync_copy(buf_v, acc_sh.at[pl.ds(sid * _RPT + i * _RCH, _RCH)])
        return c

    lax.fori_loop(0, _RPT // _RCH, zb, 0)
    plsc.subcore_barrier()

    def body(t, c):
        base = wid * _EPW + t * _K
        pltpu.sync_copy(src_hbm.at[pl.ds(base, _K)], si_v)
        pltpu.sync_copy(dst_hbm.at[pl.ds(base, _K)], di_v)
        pltpu.async_copy(xs_hbm.at[si_v], rows_v, sem).wait()
        pltpu.sync_copy(rows_v, acc_sh.at[di_v], add=True)
        return c

    lax.fori_loop(0, _NCH, body, 0)
    plsc.subcore_barrier()

    def cb(i, c):
        r0 = sid * _RPT + i * _RCH
        pltpu.sync_copy(acc_sh.at[pl.ds(r0, _RCH)], buf_v)
        pltpu.sync_copy(buf_v, out_hbm.at[cid, pl.ds(r0, _RCH)])
        return c

    lax.fori_loop(0, _RPT // _RCH, cb, 0)


def _tc1_body(x_ref, w_ref, degp_ref, xs_ref, dis_ref):
    deg = jnp.sum(degp_ref[...], axis=0) + 1.0
    dis = lax.rsqrt(deg)[:, None]
    dis_ref[...] = dis
    xs_ref[...] = jnp.dot(x_ref[...], w_ref[...],
                          preferred_element_type=jnp.float32) * dis


_tc1 = pl.pallas_call(
    _tc1_body,
    out_shape=(
        jax.ShapeDtypeStruct((_N, _D), jnp.float32),
        jax.ShapeDtypeStruct((_N, 1), jnp.float32),
    ),
)


def _tc2_body(yp_ref, xs_ref, dis_ref, b_ref, hs_ref):
    d = dis_ref[...]
    y = (yp_ref[0, :_N] + yp_ref[1, :_N] + xs_ref[...]) * d + b_ref[...]
    hs_ref[...] = jnp.maximum(y, 0.0) * d


_tc2 = pl.pallas_call(
    _tc2_body,
    out_shape=jax.ShapeDtypeStruct((_N, _D), jnp.float32),
)


def _tc3_body(yp_ref, hs_ref, dis_ref, wmu_ref, bmu_ref, wls_ref, bls_ref,
              mu_ref, ls_ref):
    agg = (yp_ref[0, :_N] + yp_ref[1, :_N] + hs_ref[...]) * dis_ref[...]
    mu_ref[...] = jnp.dot(agg, wmu_ref[...],
                          preferred_element_type=jnp.float32) + bmu_ref[...]
    ls_ref[...] = jnp.dot(agg, wls_ref[...],
                          preferred_element_type=jnp.float32) + bls_ref[...]


_tc3 = pl.pallas_call(
    _tc3_body,
    out_shape=(
        jax.ShapeDtypeStruct((_N, 64), jnp.float32),
        jax.ShapeDtypeStruct((_N, 64), jnp.float32),
    ),
)


def kernel(x, edge_index, W1, b1, Wmu, bmu, Wls, bls):
    src = edge_index[0].astype(jnp.int32)
    dst = edge_index[1].astype(jnp.int32)
    degp = _sc_deg(dst)
    xs, dis = _tc1(x, W1, degp)
    yp = _sc_spmm(xs, src, dst)
    hs = _tc2(yp, xs, dis, b1.reshape(1, -1))
    y2p = _sc_spmm(hs, src, dst)
    mu, ls = _tc3(y2p, hs, dis, Wmu, bmu.reshape(1, -1), Wls,
                  bls.reshape(1, -1))
    return (mu, ls)


# speedup vs baseline: 35.2911x; 2.0392x over previous
import functools

import jax
import jax.numpy as jnp
from jax import lax
from jax.experimental import pallas as pl
from jax.experimental.pallas import tpu as pltpu
from jax.experimental.pallas import tpu_sc as plsc

_N = 10000
_E = 320000
_D = 128
_NC = 2
_NS = 16
_NW = _NC * _NS
_K = 128
_CPW = 80
_EP = _NW * _CPW * _K
_NP = 10240
_PAD = 10200
_EPW = _E // _NW
_RPT = _NP // _NS
_RCH = 32

_mesh = plsc.VectorSubcoreMesh(core_axis_name="c", subcore_axis_name="s")


@functools.partial(
    pl.kernel,
    out_type=jax.ShapeDtypeStruct((_NW, _N), jnp.float32),
    mesh=_mesh,
    scratch_types=[
        pltpu.VMEM((_EPW,), jnp.int32),
        pltpu.VMEM((_N,), jnp.float32),
    ],
    compiler_params=pltpu.CompilerParams(needs_layout_passes=False),
)
def _sc_deg(dst_hbm, out_hbm, dst_v, hist_v):
    cid = lax.axis_index("c")
    sid = lax.axis_index("s")
    wid = sid * _NC + cid
    pltpu.sync_copy(dst_hbm.at[pl.ds(wid * _EPW, _EPW)], dst_v)
    zeros = jnp.zeros((16,), jnp.float32)
    ones = jnp.ones((16,), jnp.float32)

    def zbody(i, c):
        hist_v[pl.ds(i * 16, 16)] = zeros
        return c

    lax.fori_loop(0, _N // 16, zbody, 0)

    def hbody(i, c):
        idx = dst_v[pl.ds(i * 16, 16)]
        plsc.addupdate_scatter(hist_v, [idx], ones)
        return c

    lax.fori_loop(0, _EPW // 16, hbody, 0)
    pltpu.sync_copy(hist_v, out_hbm.at[wid])


@functools.partial(
    pl.kernel,
    out_type=jax.ShapeDtypeStruct((_NC, _NP, _D), jnp.float32),
    mesh=_mesh,
    scratch_types=[
        pltpu.VMEM((2, _K), jnp.int32),
        pltpu.VMEM((2, _K), jnp.int32),
        pltpu.VMEM((_K, _D), jnp.float32),
        pltpu.VMEM((_K, _D), jnp.float32),
        pltpu.VMEM((_RCH, _D), jnp.float32),
        pltpu.VMEM_SHARED((_NP, _D), jnp.float32),
        pltpu.SemaphoreType.DMA,
        pltpu.SemaphoreType.DMA,
        pltpu.SemaphoreType.DMA,
        pltpu.SemaphoreType.DMA,
    ],
)
def _sc_spmm(xs_hbm, sd_hbm, out_hbm, sd0_v, sd1_v, r0_v, r1_v,
             buf_v, acc_sh, semg0, semg1, semi0, semi1):
    cid = lax.axis_index("c")
    sid = lax.axis_index("s")
    wid = sid * _NC + cid
    row0 = wid * _CPW

    zeros = jnp.zeros((16,), jnp.float32)
    for r in range(_RCH):
        for j in range(_D // 16):
            buf_v[r, pl.ds(j * 16, 16)] = zeros

    def zb(i, c):
        pltpu.sync_copy(buf_v, acc_sh.at[pl.ds(sid * _RPT + i * _RCH, _RCH)])
        return c

    lax.fori_loop(0, _RPT // _RCH, zb, 0)
    plsc.subcore_barrier()

    pltpu.sync_copy(sd_hbm.at[row0], sd0_v)
    pltpu.async_copy(sd_hbm.at[row0 + 1], sd1_v, semi1)
    pltpu.async_copy(xs_hbm.at[sd0_v.at[0]], r0_v, semg0)

    def body(i, c):
        t = row0 + i * 2
        pltpu.make_async_copy(xs_hbm.at[sd0_v.at[0]], r0_v, semg0).wait()
        pltpu.make_async_copy(sd_hbm.at[t + 1], sd1_v, semi1).wait()
        pltpu.async_copy(xs_hbm.at[sd1_v.at[0]], r1_v, semg1)
        pltpu.sync_copy(r0_v, acc_sh.at[sd0_v.at[1]], add=True)
        pltpu.async_copy(sd_hbm.at[t + 2], sd0_v, semi0)
        pltpu.make_async_copy(xs_hbm.at[sd1_v.at[0]], r1_v, semg1).wait()
        pltpu.make_async_copy(sd_hbm.at[t + 2], sd0_v, semi0).wait()
        pltpu.async_copy(xs_hbm.at[sd0_v.at[0]], r0_v, semg0)
        pltpu.sync_copy(r1_v, acc_sh.at[sd1_v.at[1]], add=True)
        pltpu.async_copy(sd_hbm.at[t + 3], sd1_v, semi1)
        return c

    lax.fori_loop(0, _CPW // 2 - 1, body, 0)
    pltpu.make_async_copy(xs_hbm.at[sd0_v.at[0]], r0_v, semg0).wait()
    pltpu.make_async_copy(sd_hbm.at[row0 + _CPW - 1], sd1_v, semi1).wait()
    pltpu.async_copy(xs_hbm.at[sd1_v.at[0]], r1_v, semg1)
    pltpu.sync_copy(r0_v, acc_sh.at[sd0_v.at[1]], add=True)
    pltpu.make_async_copy(xs_hbm.at[sd1_v.at[0]], r1_v, semg1).wait()
    pltpu.sync_copy(r1_v, acc_sh.at[sd1_v.at[1]], add=True)

    plsc.subcore_barrier()

    def cb(i, c):
        r0 = sid * _RPT + i * _RCH
        pltpu.sync_copy(acc_sh.at[pl.ds(r0, _RCH)], buf_v)
        pltpu.sync_copy(buf_v, out_hbm.at[cid, pl.ds(r0, _RCH)])
        return c

    lax.fori_loop(0, _RPT // _RCH, cb, 0)


def _tc1_body(x_ref, w_ref, degp_ref, xs_ref, dis_ref):
    deg = jnp.sum(degp_ref[...], axis=0) + 1.0
    dis = lax.rsqrt(deg)[:, None]
    dis_ref[...] = dis
    xs_ref[:_N] = jnp.dot(x_ref[...], w_ref[...],
                          preferred_element_type=jnp.float32) * dis
    xs_ref[_N:] = jnp.zeros((_NP - _N, _D), jnp.float32)


_tc1 = pl.pallas_call(
    _tc1_body,
    out_shape=(
        jax.ShapeDtypeStruct((_NP, _D), jnp.float32),
        jax.ShapeDtypeStruct((_N, 1), jnp.float32),
    ),
)


def _tc2_body(yp_ref, xs_ref, dis_ref, b_ref, hs_ref):
    d = dis_ref[...]
    y = (yp_ref[0, :_N] + yp_ref[1, :_N] + xs_ref[:_N]) * d + b_ref[...]
    hs_ref[:_N] = jnp.maximum(y, 0.0) * d
    hs_ref[_N:] = jnp.zeros((_NP - _N, _D), jnp.float32)


_tc2 = pl.pallas_call(
    _tc2_body,
    out_shape=jax.ShapeDtypeStruct((_NP, _D), jnp.float32),
)


def _tc3_body(yp_ref, hs_ref, dis_ref, wmu_ref, bmu_ref, wls_ref, bls_ref,
              mu_ref, ls_ref):
    agg = (yp_ref[0, :_N] + yp_ref[1, :_N] + hs_ref[:_N]) * dis_ref[...]
    mu_ref[...] = jnp.dot(agg, wmu_ref[...],
                          preferred_element_type=jnp.float32) + bmu_ref[...]
    ls_ref[...] = jnp.dot(agg, wls_ref[...],
                          preferred_element_type=jnp.float32) + bls_ref[...]


_tc3 = pl.pallas_call(
    _tc3_body,
    out_shape=(
        jax.ShapeDtypeStruct((_N, 64), jnp.float32),
        jax.ShapeDtypeStruct((_N, 64), jnp.float32),
    ),
)


def kernel(x, edge_index, W1, b1, Wmu, bmu, Wls, bls):
    src = edge_index[0].astype(jnp.int32)
    dst = edge_index[1].astype(jnp.int32)
    pad = _N + jnp.arange(_EP - _E, dtype=jnp.int32) % (_NP - _N)
    srcp = jnp.concatenate([src, pad]).reshape(_EP // _K, _K)
    dstp = jnp.concatenate([dst, pad]).reshape(_EP // _K, _K)
    sd = jnp.stack([srcp, dstp], axis=1)
    degp = _sc_deg(dst)
    xs, dis = _tc1(x, W1, degp)
    yp = _sc_spmm(xs, sd)
    hs = _tc2(yp, xs, dis, b1.reshape(1, -1))
    y2p = _sc_spmm(hs, sd)
    mu, ls = _tc3(y2p, hs, dis, Wmu, bmu.reshape(1, -1), Wls,
                  bls.reshape(1, -1))
    return (mu, ls)


# speedup vs baseline: 36.1037x; 1.0230x over previous
import functools

import jax
import jax.numpy as jnp
from jax import lax
from jax.experimental import pallas as pl
from jax.experimental.pallas import tpu as pltpu
from jax.experimental.pallas import tpu_sc as plsc

_N = 10000
_E = 320000
_D = 128
_NC = 2
_NS = 16
_NW = _NC * _NS
_K = 128
_CPW = 80
_EP = _NW * _CPW * _K
_NP = 10240
_PAD = 10200
_EPW = _E // _NW
_RPT = _NP // _NS
_RCH = 32

_mesh = plsc.VectorSubcoreMesh(core_axis_name="c", subcore_axis_name="s")


@functools.partial(
    pl.kernel,
    out_type=jax.ShapeDtypeStruct((_NW, _N), jnp.float32),
    mesh=_mesh,
    scratch_types=[
        pltpu.VMEM((_EPW,), jnp.int32),
        pltpu.VMEM((_N,), jnp.float32),
    ],
    compiler_params=pltpu.CompilerParams(needs_layout_passes=False),
)
def _sc_deg(dst_hbm, out_hbm, dst_v, hist_v):
    cid = lax.axis_index("c")
    sid = lax.axis_index("s")
    wid = sid * _NC + cid
    pltpu.sync_copy(dst_hbm.at[pl.ds(wid * _EPW, _EPW)], dst_v)
    zeros = jnp.zeros((16,), jnp.float32)
    ones = jnp.ones((16,), jnp.float32)

    def zbody(i, c):
        hist_v[pl.ds(i * 16, 16)] = zeros
        return c

    lax.fori_loop(0, _N // 16, zbody, 0)

    def hbody(i, c):
        idx = dst_v[pl.ds(i * 16, 16)]
        plsc.addupdate_scatter(hist_v, [idx], ones)
        return c

    lax.fori_loop(0, _EPW // 16, hbody, 0)
    pltpu.sync_copy(hist_v, out_hbm.at[wid])


@functools.partial(
    pl.kernel,
    out_type=jax.ShapeDtypeStruct((_NC, _NP, _D), jnp.float32),
    mesh=_mesh,
    scratch_types=[
        pltpu.VMEM((2, _K), jnp.int32),
        pltpu.VMEM((2, _K), jnp.int32),
        pltpu.VMEM((_K, _D), jnp.float32),
        pltpu.VMEM((_K, _D), jnp.float32),
        pltpu.VMEM((_RCH, _D), jnp.float32),
        pltpu.VMEM((_RCH, _D), jnp.float32),
        pltpu.VMEM_SHARED((_NP, _D), jnp.float32),
        pltpu.SemaphoreType.DMA,
        pltpu.SemaphoreType.DMA,
        pltpu.SemaphoreType.DMA,
        pltpu.SemaphoreType.DMA,
        pltpu.SemaphoreType.DMA,
    ],
)
def _sc_spmm(xs_hbm, sd_hbm, out_hbm, sd0_v, sd1_v, r0_v, r1_v,
             bufa_v, bufb_v, acc_sh, semg0, semg1, semi0, semi1, semz):
    cid = lax.axis_index("c")
    sid = lax.axis_index("s")
    wid = sid * _NC + cid
    row0 = wid * _CPW
    rbase = sid * _RPT

    pltpu.async_copy(sd_hbm.at[row0], sd0_v, semi0)
    pltpu.async_copy(sd_hbm.at[row0 + 1], sd1_v, semi1)
    zeros = jnp.zeros((16,), jnp.float32)
    for r in range(_RCH):
        for j in range(_D // 16):
            bufa_v[r, pl.ds(j * 16, 16)] = zeros
    pltpu.make_async_copy(sd_hbm.at[row0], sd0_v, semi0).wait()
    pltpu.async_copy(xs_hbm.at[sd0_v.at[0]], r0_v, semg0)

    def zb(i, c):
        pltpu.async_copy(bufa_v, acc_sh.at[pl.ds(rbase + i * _RCH, _RCH)],
                         semz)
        return c

    lax.fori_loop(0, _RPT // _RCH, zb, 0)

    def zw(i, c):
        pltpu.make_async_copy(bufa_v, acc_sh.at[pl.ds(rbase + i * _RCH,
                                                      _RCH)], semz).wait()
        return c

    lax.fori_loop(0, _RPT // _RCH, zw, 0)
    plsc.subcore_barrier()


    def body(i, c):
        t = row0 + i * 2
        pltpu.make_async_copy(xs_hbm.at[sd0_v.at[0]], r0_v, semg0).wait()
        pltpu.make_async_copy(sd_hbm.at[t + 1], sd1_v, semi1).wait()
        pltpu.async_copy(xs_hbm.at[sd1_v.at[0]], r1_v, semg1)
        pltpu.sync_copy(r0_v, acc_sh.at[sd0_v.at[1]], add=True)
        pltpu.async_copy(sd_hbm.at[t + 2], sd0_v, semi0)
        pltpu.make_async_copy(xs_hbm.at[sd1_v.at[0]], r1_v, semg1).wait()
        pltpu.make_async_copy(sd_hbm.at[t + 2], sd0_v, semi0).wait()
        pltpu.async_copy(xs_hbm.at[sd0_v.at[0]], r0_v, semg0)
        pltpu.sync_copy(r1_v, acc_sh.at[sd1_v.at[1]], add=True)
        pltpu.async_copy(sd_hbm.at[t + 3], sd1_v, semi1)
        return c

    lax.fori_loop(0, _CPW // 2 - 1, body, 0)
    pltpu.make_async_copy(xs_hbm.at[sd0_v.at[0]], r0_v, semg0).wait()
    pltpu.make_async_copy(sd_hbm.at[row0 + _CPW - 1], sd1_v, semi1).wait()
    pltpu.async_copy(xs_hbm.at[sd1_v.at[0]], r1_v, semg1)
    pltpu.sync_copy(r0_v, acc_sh.at[sd0_v.at[1]], add=True)
    pltpu.make_async_copy(xs_hbm.at[sd1_v.at[0]], r1_v, semg1).wait()
    pltpu.sync_copy(r1_v, acc_sh.at[sd1_v.at[1]], add=True)

    plsc.subcore_barrier()

    def _fill(t, buf, sem):
        pltpu.async_copy(acc_sh.at[pl.ds(rbase + t * _RCH, _RCH)], buf, sem)

    def _fillw(t, buf, sem):
        pltpu.make_async_copy(acc_sh.at[pl.ds(rbase + t * _RCH, _RCH)], buf,
                              sem).wait()

    def _drain(t, buf, sem):
        pltpu.async_copy(buf, out_hbm.at[cid, pl.ds(rbase + t * _RCH, _RCH)],
                         sem)

    def _drainw(t, buf, sem):
        pltpu.make_async_copy(buf, out_hbm.at[cid, pl.ds(rbase + t * _RCH,
                                                         _RCH)], sem).wait()

    nco = _RPT // _RCH
    _fill(0, bufa_v, semg0)
    _fill(1, bufb_v, semg1)

    def cb(i, c):
        t = i * 2
        _fillw(t, bufa_v, semg0)
        _drain(t, bufa_v, semi0)
        _fillw(t + 1, bufb_v, semg1)
        _drain(t + 1, bufb_v, semi1)
        _drainw(t, bufa_v, semi0)
        _fill(t + 2, bufa_v, semg0)
        _drainw(t + 1, bufb_v, semi1)
        _fill(t + 3, bufb_v, semg1)
        return c

    lax.fori_loop(0, nco // 2 - 1, cb, 0)
    _fillw(nco - 2, bufa_v, semg0)
    _drain(nco - 2, bufa_v, semi0)
    _fillw(nco - 1, bufb_v, semg1)
    _drain(nco - 1, bufb_v, semi1)
    _drainw(nco - 2, bufa_v, semi0)
    _drainw(nco - 1, bufb_v, semi1)


def _tc1_body(x_ref, w_ref, degp_ref, xs_ref, dis_ref):
    deg = jnp.sum(degp_ref[...], axis=0) + 1.0
    dis = lax.rsqrt(deg)[:, None]
    dis_ref[...] = dis
    xs_ref[:_N] = jnp.dot(x_ref[...], w_ref[...],
                          preferred_element_type=jnp.float32) * dis
    xs_ref[_N:] = jnp.zeros((_NP - _N, _D), jnp.float32)


_tc1 = pl.pallas_call(
    _tc1_body,
    out_shape=(
        jax.ShapeDtypeStruct((_NP, _D), jnp.float32),
        jax.ShapeDtypeStruct((_N, 1), jnp.float32),
    ),
)


def _tc2_body(yp_ref, xs_ref, dis_ref, b_ref, hs_ref):
    d = dis_ref[...]
    y = (yp_ref[0, :_N] + yp_ref[1, :_N] + xs_ref[:_N]) * d + b_ref[...]
    hs_ref[:_N] = jnp.maximum(y, 0.0) * d
    hs_ref[_N:] = jnp.zeros((_NP - _N, _D), jnp.float32)


_tc2 = pl.pallas_call(
    _tc2_body,
    out_shape=jax.ShapeDtypeStruct((_NP, _D), jnp.float32),
)


def _tc3_body(yp_ref, hs_ref, dis_ref, wmu_ref, bmu_ref, wls_ref, bls_ref,
              mu_ref, ls_ref):
    agg = (yp_ref[0, :_N] + yp_ref[1, :_N] + hs_ref[:_N]) * dis_ref[...]
    mu_ref[...] = jnp.dot(agg, wmu_ref[...],
                          preferred_element_type=jnp.float32) + bmu_ref[...]
    ls_ref[...] = jnp.dot(agg, wls_ref[...],
                          preferred_element_type=jnp.float32) + bls_ref[...]


_tc3 = pl.pallas_call(
    _tc3_body,
    out_shape=(
        jax.ShapeDtypeStruct((_N, 64), jnp.float32),
        jax.ShapeDtypeStruct((_N, 64), jnp.float32),
    ),
)


def kernel(x, edge_index, W1, b1, Wmu, bmu, Wls, bls):
    src = edge_index[0].astype(jnp.int32)
    dst = edge_index[1].astype(jnp.int32)
    pad = _N + jnp.arange(_EP - _E, dtype=jnp.int32) % (_NP - _N)
    srcp = jnp.concatenate([src, pad]).reshape(_EP // _K, _K)
    dstp = jnp.concatenate([dst, pad]).reshape(_EP // _K, _K)
    sd = jnp.stack([srcp, dstp], axis=1)
    degp = _sc_deg(dst)
    xs, dis = _tc1(x, W1, degp)
    yp = _sc_spmm(xs, sd)
    hs = _tc2(yp, xs, dis, b1.reshape(1, -1))
    y2p = _sc_spmm(hs, sd)
    mu, ls = _tc3(y2p, hs, dis, Wmu, bmu.reshape(1, -1), Wls,
                  bls.reshape(1, -1))
    return (mu, ls)
